# baseline (speedup 1.0000x reference)
import jax
import jax.numpy as jnp
from jax import lax
from jax.experimental import pallas as pl
from jax.experimental.pallas import tpu as pltpu
from jax.experimental.pallas import tpu_sc as plsc

_N = 32768
_B = 16
_D = 256
_NC = 2
_NS = 16
_L = 16
_C = 256
_H = _D // _NC
_HV = _H // _L


def _sum_body(x_hbm, len_hbm, out_hbm, len_v, buf0, buf1, acc, rows16,
              shared, sem0, sem1):
    cid = lax.axis_index("c")
    sid = lax.axis_index("s")
    col0 = cid * _H

    pltpu.sync_copy(len_hbm, len_v)
    lens = len_v[...]
    csum = plsc.cumsum(lens)
    total = jnp.max(csum)
    lane = lax.iota(jnp.int32, _L)

    zero = jnp.zeros((_L,), jnp.float32)

    def zbody(i, c):
        for j in range(_HV):
            acc[i, pl.ds(j * _L, _L)] = zero
        return c

    lax.fori_loop(0, _B, zbody, 0)

    nchunks = (total + _C - 1) // _C
    kw = (nchunks - sid + _NS - 1) // _NS

    bufs = (buf0, buf1)
    sems = (sem0, sem1)

    def copy_of(k, slot):
        row0 = (sid + k * _NS) * _C
        return pltpu.make_async_copy(
            x_hbm.at[pl.ds(row0, _C), pl.ds(col0, _H)], bufs[slot],
            sems[slot]
        )

    @pl.when(kw > 0)
    def _():
        copy_of(0, 0).start()

    def process(k, slot):
        buf = bufs[slot]
        row0 = (sid + k * _NS) * _C

        @pl.when(k + 1 < kw)
        def _():
            copy_of(k + 1, 1 - slot).start()

        copy_of(k, slot).wait()
        row1 = jnp.minimum(row0 + _C, total)
        s0 = jnp.sum(jnp.where(csum <= row0, 1, 0))

        def run_cond(st):
            return st[1] < row1

        def run_body(st):
            s, a = st
            end_s = jnp.max(jnp.where(lane == s, csum, 0))
            b = jnp.minimum(end_s, row1)

            def rbody(rr, carry):
                return tuple(
                    carry[j] + buf[rr, pl.ds(j * _L, _L)]
                    for j in range(_HV)
                )

            run = lax.fori_loop(a - row0, b - row0, rbody, (zero,) * _HV)
            for j in range(_HV):
                o = j * _L
                acc[s, pl.ds(o, _L)] = acc[s, pl.ds(o, _L)] + run[j]
            return (s + 1, b)

        lax.while_loop(run_cond, run_body, (s0, row0))

    def pair_body(i, c):
        k = i * 2
        for slot in range(2):
            @pl.when(k + slot < kw)
            def _():
                process(k + slot, slot)
        return c

    lax.fori_loop(0, (kw + 1) // 2, pair_body, 0)

    for s in range(_B):
        pltpu.sync_copy(acc.at[s], shared.at[s, sid])
    plsc.subcore_barrier()
    plsc.subcore_barrier()

    pltpu.sync_copy(shared.at[sid], rows16)

    def fbody(w, carry):
        return tuple(
            carry[j] + rows16[w, pl.ds(j * _L, _L)] for j in range(_HV)
        )

    tot = lax.fori_loop(0, _NS, fbody, (zero,) * _HV)
    cnt = jnp.max(jnp.where(lane == sid, jnp.maximum(lens, 1), 0))
    cnt_vec = jnp.full((_L,), cnt, jnp.int32).astype(jnp.float32)
    recip = jnp.ones((_L,), jnp.float32) / cnt_vec
    for j in range(_HV):
        rows16[0, pl.ds(j * _L, _L)] = tot[j] * recip
    pltpu.sync_copy(rows16.at[0], out_hbm.at[sid, pl.ds(col0, _H)])


_mesh = plsc.VectorSubcoreMesh(core_axis_name="c", subcore_axis_name="s")
_params = pltpu.CompilerParams(needs_layout_passes=False)

_sum_call = pl.kernel(
    _sum_body,
    out_type=jax.ShapeDtypeStruct((_B, _D), jnp.float32),
    mesh=_mesh,
    compiler_params=_params,
    scratch_types=[
        pltpu.VMEM((_L,), jnp.int32),
        pltpu.VMEM((_C, _H), jnp.float32),
        pltpu.VMEM((_C, _H), jnp.float32),
        pltpu.VMEM((_B, _H), jnp.float32),
        pltpu.VMEM((_NS, _H), jnp.float32),
        pltpu.VMEM_SHARED((_B, _NS, _H), jnp.float32),
        pltpu.SemaphoreType.DMA,
        pltpu.SemaphoreType.DMA,
    ],
)


def kernel(x, batch_lengths):
    return _sum_call(x, batch_lengths)
